# baseline (speedup 1.0000x reference)
import functools

import jax
import jax.numpy as jnp
from jax import lax
from jax.experimental import pallas as pl
from jax.experimental.pallas import tpu as pltpu
from jax.experimental.pallas import tpu_sc as plsc

N = 10000
E = 320000
F = 128
NC, NS = 2, 16
NW = NC * NS
EPW = E // NW
K = 96
NCHUNK = EPW // K
TAIL = EPW - NCHUNK * K
NP = 10240
RPT = NP // NS
ZR = 32

_mesh = plsc.VectorSubcoreMesh(
    core_axis_name="c", subcore_axis_name="s", num_cores=NC, num_subcores=NS
)


@functools.partial(
    pl.kernel,
    out_type=jax.ShapeDtypeStruct((NC, NP), jnp.float32),
    mesh=_mesh,
    scratch_types=[
        pltpu.VMEM((RPT,), jnp.float32),
        pltpu.VMEM((K,), jnp.float32),
        pltpu.VMEM((TAIL,), jnp.float32),
        pltpu.VMEM((NCHUNK, K), jnp.int32),
        pltpu.VMEM((TAIL,), jnp.int32),
        pltpu.VMEM_SHARED((NP,), jnp.float32),
    ],
)
def _sc_deg(dst_m_hbm, etail_hbm, out_hbm, zbuf, ones_v, ones_t,
            dst_all, dst_t, acc):
    cid = lax.axis_index("c")
    sid = lax.axis_index("s")
    wid = cid * NS + sid

    @pl.loop(0, RPT // 16)
    def _(i):
        zbuf[pl.ds(i * 16, 16)] = jnp.zeros((16,), jnp.float32)

    @pl.loop(0, K // 16)
    def _(i):
        ones_v[pl.ds(i * 16, 16)] = jnp.ones((16,), jnp.float32)

    ones_t[...] = jnp.ones((TAIL,), jnp.float32)

    pltpu.sync_copy(dst_m_hbm.at[wid], dst_all)
    pltpu.sync_copy(etail_hbm.at[wid, 1], dst_t)
    pltpu.sync_copy(zbuf, acc.at[pl.ds(sid * RPT, RPT)])
    plsc.subcore_barrier()

    @pl.loop(0, NCHUNK)
    def _(j):
        pltpu.sync_copy(ones_v, acc.at[dst_all.at[j]], add=True)

    pltpu.sync_copy(ones_t, acc.at[dst_t], add=True)

    plsc.subcore_barrier()
    pltpu.sync_copy(
        acc.at[pl.ds(sid * RPT, RPT)], out_hbm.at[cid, pl.ds(sid * RPT, RPT)]
    )


@functools.partial(
    pl.kernel,
    out_type=jax.ShapeDtypeStruct((NC, NP, F), jnp.float32),
    mesh=_mesh,
    scratch_types=[
        pltpu.VMEM((ZR, F), jnp.float32),
        pltpu.VMEM((2, K), jnp.int32),
        pltpu.VMEM((2, K), jnp.int32),
        pltpu.VMEM((2, K), jnp.int32),
        pltpu.VMEM((2, K), jnp.int32),
        pltpu.VMEM((2, K), jnp.int32),
        pltpu.VMEM((2, K), jnp.int32),
        pltpu.VMEM((K, F), jnp.float32),
        pltpu.VMEM((K, F), jnp.float32),
        pltpu.VMEM((K, F), jnp.float32),
        pltpu.VMEM((2, TAIL), jnp.int32),
        pltpu.VMEM((TAIL, F), jnp.float32),
        pltpu.SemaphoreType.DMA,
        pltpu.SemaphoreType.DMA,
        pltpu.SemaphoreType.DMA,
        pltpu.SemaphoreType.DMA,
        pltpu.SemaphoreType.DMA,
        pltpu.SemaphoreType.DMA,
        pltpu.SemaphoreType.DMA,
        pltpu.SemaphoreType.DMA,
        pltpu.SemaphoreType.DMA,
        pltpu.SemaphoreType.DMA,
        pltpu.SemaphoreType.DMA,
        pltpu.SemaphoreType.DMA,
        pltpu.VMEM_SHARED((NP, F), jnp.float32),
    ],
)
def _sc_agg(g_hbm, eidx_hbm, etail_hbm, out_hbm,
            zbuf, is0, is1, is2, is3, is4, is5, rb0, rb1, rb2, idx_t, rows_t,
            gs0, gs1, gs2, ss0, ss1, ss2, ism0, ism1, ism2, ism3, ism4, ism5,
            acc):
    cid = lax.axis_index("c")
    sid = lax.axis_index("s")
    wid = cid * NS + sid

    @pl.loop(0, ZR)
    def _(r):
        for c2 in range(F // 16):
            zbuf[r, pl.ds(c2 * 16, 16)] = jnp.zeros((16,), jnp.float32)

    @pl.loop(0, RPT // ZR)
    def _(i):
        pltpu.async_copy(zbuf, acc.at[pl.ds(sid * RPT + i * ZR, ZR)], gs0)

    @pl.loop(0, RPT // ZR)
    def _(i):
        pltpu.make_async_copy(zbuf, acc.at[pl.ds(sid * RPT, ZR)], gs0).wait()

    plsc.subcore_barrier()

    cbase = wid * NCHUNK
    slots = ((is0, ism0), (is1, ism1), (is2, ism2),
             (is3, ism3), (is4, ism4), (is5, ism5))
    rows = ((rb0, gs0), (rb1, gs1), (rb2, gs2))
    ssems = (ss0, ss1, ss2)

    def _chunk_step(c0, q, first=False):
        idx_r, _ = slots[q]
        gidx_r, gisem_r = slots[(q + 2) % 6]
        pidx_r, pisem_r = slots[(q + 5) % 6]
        rows_r, gsem_r = rows[q % 3]
        grows_r, ggsem_r = rows[(q + 2) % 3]
        pltpu.make_async_copy(g_hbm.at[idx_r.at[0]], rows_r, gsem_r).wait()
        pltpu.async_copy(rows_r, acc.at[idx_r.at[1]], ssems[q % 3], add=True)
        pltpu.make_async_copy(eidx_hbm.at[cbase], gidx_r, gisem_r).wait()
        if not first:
            pltpu.make_async_copy(
                grows_r, acc.at[idx_r.at[1]], ssems[(q + 2) % 3]).wait()
        pltpu.async_copy(g_hbm.at[gidx_r.at[0]], grows_r, ggsem_r)
        pltpu.async_copy(eidx_hbm.at[cbase + c0 + q + 5], pidx_r, pisem_r)

    pltpu.sync_copy(eidx_hbm.at[cbase + 0], is0)
    pltpu.sync_copy(eidx_hbm.at[cbase + 1], is1)
    for q in range(2, 5):
        pltpu.async_copy(eidx_hbm.at[cbase + q], slots[q][0], slots[q][1])
    pltpu.async_copy(g_hbm.at[is0.at[0]], rb0, gs0)
    pltpu.async_copy(g_hbm.at[is1.at[0]], rb1, gs1)

    _chunk_step(0, 0, first=True)
    for q in range(1, 6):
        _chunk_step(0, q)

    @pl.loop(1, (NCHUNK - 2) // 6)
    def _(t):
        c0 = 6 * t
        for q in range(6):
            _chunk_step(c0, q)

    for q in range(2):
        idx_r, _ = slots[q]
        rows_r, gsem_r = rows[q]
        pltpu.make_async_copy(g_hbm.at[idx_r.at[0]], rows_r, gsem_r).wait()
        pltpu.async_copy(rows_r, acc.at[idx_r.at[1]], ssems[q], add=True)
    for b in range(3):
        pltpu.make_async_copy(
            rows[b][0], acc.at[slots[b][0].at[1]], ssems[b]).wait()
    for q in range(2, 5):
        pltpu.make_async_copy(eidx_hbm.at[cbase], slots[q][0], slots[q][1]).wait()

    pltpu.sync_copy(etail_hbm.at[wid], idx_t)
    pltpu.async_copy(g_hbm.at[idx_t.at[0]], rows_t, gs0).wait()
    pltpu.sync_copy(rows_t, acc.at[idx_t.at[1]], add=True)

    plsc.subcore_barrier()
    pltpu.sync_copy(
        acc.at[pl.ds(sid * RPT, RPT)], out_hbm.at[cid, pl.ds(sid * RPT, RPT)]
    )


BM = 1000

_CONTRACT_T = (((1,), (1,)), ((), ()))


def _tc1_body(d0, d1, x, w1, g_out, dinv_out):
    deg = d0[...] + d1[...] + 1.0
    dinv = lax.rsqrt(deg)
    h = lax.dot_general(x[...], w1[...], _CONTRACT_T,
                        preferred_element_type=jnp.float32)
    g_out[...] = dinv * h
    dinv_out[...] = dinv


_tc1 = pl.pallas_call(
    _tc1_body,
    grid=(N // BM,),
    in_specs=[
        pl.BlockSpec((BM, 1), lambda r: (r, 0)),
        pl.BlockSpec((BM, 1), lambda r: (r, 0)),
        pl.BlockSpec((BM, F), lambda r: (r, 0)),
        pl.BlockSpec((F, F), lambda r: (0, 0)),
    ],
    out_specs=[
        pl.BlockSpec((BM, F), lambda r: (r, 0)),
        pl.BlockSpec((BM, 1), lambda r: (r, 0)),
    ],
    out_shape=[
        jax.ShapeDtypeStruct((N, F), jnp.float32),
        jax.ShapeDtypeStruct((N, 1), jnp.float32),
    ],
)


def _tc2_body(acc, g1, dinv, b1, alpha, w2, g2_out):
    h = dinv[...] * (acc[0] + acc[1] + g1[...]) + b1[...]
    p = jnp.where(h > 0, h, alpha[...] * h)
    m = lax.dot_general(p, w2[...], _CONTRACT_T,
                        preferred_element_type=jnp.float32)
    g2_out[...] = dinv[...] * m


_tc2 = pl.pallas_call(
    _tc2_body,
    grid=(N // BM,),
    in_specs=[
        pl.BlockSpec((NC, BM, F), lambda r: (0, r, 0)),
        pl.BlockSpec((BM, F), lambda r: (r, 0)),
        pl.BlockSpec((BM, 1), lambda r: (r, 0)),
        pl.BlockSpec((1, F), lambda r: (0, 0)),
        pl.BlockSpec((1, F), lambda r: (0, 0)),
        pl.BlockSpec((F, F), lambda r: (0, 0)),
    ],
    out_specs=pl.BlockSpec((BM, F), lambda r: (r, 0)),
    out_shape=jax.ShapeDtypeStruct((N, F), jnp.float32),
)


def _tc3_body(acc, g2, dinv, b2, out):
    out[...] = dinv[...] * (acc[0] + acc[1] + g2[...]) + b2[...]


_tc3 = pl.pallas_call(
    _tc3_body,
    grid=(N // BM,),
    in_specs=[
        pl.BlockSpec((NC, BM, F), lambda r: (0, r, 0)),
        pl.BlockSpec((BM, F), lambda r: (r, 0)),
        pl.BlockSpec((BM, 1), lambda r: (r, 0)),
        pl.BlockSpec((1, F), lambda r: (0, 0)),
    ],
    out_specs=pl.BlockSpec((BM, F), lambda r: (r, 0)),
    out_shape=jax.ShapeDtypeStruct((N, F), jnp.float32),
)


def kernel(x, edge_index, W1, b1, W2, b2, alpha):
    ei = edge_index.reshape(2, NW, EPW)
    src_m = ei[0, :, : NCHUNK * K].reshape(NW, NCHUNK, K)
    dst_m = ei[1, :, : NCHUNK * K].reshape(NW, NCHUNK, K)
    eidx = jnp.concatenate(
        [jnp.stack([src_m, dst_m], axis=2).reshape(NW * NCHUNK, 2, K),
         jnp.zeros((4, 2, K), jnp.int32)])
    etail = ei[:, :, NCHUNK * K :].transpose(1, 0, 2)

    degp = _sc_deg(dst_m, etail)
    g1, dinv = _tc1(degp[0].reshape(NP, 1)[:N],
                    degp[1].reshape(NP, 1)[:N], x, W1)
    acc1 = _sc_agg(g1, eidx, etail)
    g2 = _tc2(acc1, g1, dinv, b1.reshape(1, F), alpha.reshape(1, F), W2)
    acc2 = _sc_agg(g2, eidx, etail)
    out = _tc3(acc2, g2, dinv, b2.reshape(1, F))
    return out


# speedup vs baseline: 1.0474x; 1.0474x over previous
import functools

import jax
import jax.numpy as jnp
from jax import lax
from jax.experimental import pallas as pl
from jax.experimental.pallas import tpu as pltpu
from jax.experimental.pallas import tpu_sc as plsc

N = 10000
E = 320000
F = 128
NC, NS = 2, 16
NW = NC * NS
EPW = E // NW
K = 96
NCHUNK = EPW // K
TAIL = EPW - NCHUNK * K
NP = 10240
RPT = NP // NS
ZR = 32

_mesh = plsc.VectorSubcoreMesh(
    core_axis_name="c", subcore_axis_name="s", num_cores=NC, num_subcores=NS
)


@functools.partial(
    pl.kernel,
    out_type=jax.ShapeDtypeStruct((NC, NP), jnp.float32),
    mesh=_mesh,
    scratch_types=[
        pltpu.VMEM((RPT,), jnp.float32),
        pltpu.VMEM((K,), jnp.float32),
        pltpu.VMEM((TAIL,), jnp.float32),
        pltpu.VMEM((NCHUNK, K), jnp.int32),
        pltpu.VMEM((TAIL,), jnp.int32),
        pltpu.SemaphoreType.DMA,
        pltpu.VMEM_SHARED((NP,), jnp.float32),
    ],
)
def _sc_deg(dst_m_hbm, etail_hbm, out_hbm, zbuf, ones_v, ones_t,
            dst_all, dst_t, dsem, acc):
    cid = lax.axis_index("c")
    sid = lax.axis_index("s")
    wid = cid * NS + sid

    @pl.loop(0, RPT // 16)
    def _(i):
        zbuf[pl.ds(i * 16, 16)] = jnp.zeros((16,), jnp.float32)

    @pl.loop(0, K // 16)
    def _(i):
        ones_v[pl.ds(i * 16, 16)] = jnp.ones((16,), jnp.float32)

    ones_t[...] = jnp.ones((TAIL,), jnp.float32)

    pltpu.sync_copy(dst_m_hbm.at[wid], dst_all)
    pltpu.sync_copy(etail_hbm.at[wid, 1], dst_t)
    pltpu.sync_copy(zbuf, acc.at[pl.ds(sid * RPT, RPT)])
    plsc.subcore_barrier()

    @pl.loop(0, 8)
    def _(j):
        pltpu.async_copy(ones_v, acc.at[dst_all.at[j]], dsem, add=True)

    @pl.loop(8, NCHUNK)
    def _(j):
        pltpu.make_async_copy(ones_v, acc.at[dst_all.at[0]], dsem).wait()
        pltpu.async_copy(ones_v, acc.at[dst_all.at[j]], dsem, add=True)

    @pl.loop(0, 8)
    def _(j):
        pltpu.make_async_copy(ones_v, acc.at[dst_all.at[0]], dsem).wait()

    pltpu.sync_copy(ones_t, acc.at[dst_t], add=True)

    plsc.subcore_barrier()
    pltpu.sync_copy(
        acc.at[pl.ds(sid * RPT, RPT)], out_hbm.at[cid, pl.ds(sid * RPT, RPT)]
    )


@functools.partial(
    pl.kernel,
    out_type=jax.ShapeDtypeStruct((NC, NP, F), jnp.float32),
    mesh=_mesh,
    scratch_types=[
        pltpu.VMEM((ZR, F), jnp.float32),
        pltpu.VMEM((2, K), jnp.int32),
        pltpu.VMEM((2, K), jnp.int32),
        pltpu.VMEM((2, K), jnp.int32),
        pltpu.VMEM((2, K), jnp.int32),
        pltpu.VMEM((2, K), jnp.int32),
        pltpu.VMEM((2, K), jnp.int32),
        pltpu.VMEM((K, F), jnp.float32),
        pltpu.VMEM((K, F), jnp.float32),
        pltpu.VMEM((K, F), jnp.float32),
        pltpu.VMEM((2, TAIL), jnp.int32),
        pltpu.VMEM((TAIL, F), jnp.float32),
        pltpu.SemaphoreType.DMA,
        pltpu.SemaphoreType.DMA,
        pltpu.SemaphoreType.DMA,
        pltpu.SemaphoreType.DMA,
        pltpu.SemaphoreType.DMA,
        pltpu.SemaphoreType.DMA,
        pltpu.SemaphoreType.DMA,
        pltpu.SemaphoreType.DMA,
        pltpu.SemaphoreType.DMA,
        pltpu.VMEM_SHARED((NP, F), jnp.float32),
    ],
)
def _sc_agg(g_hbm, eidx_hbm, etail_hbm, out_hbm,
            zbuf, is0, is1, is2, is3, is4, is5, rb0, rb1, rb2, idx_t, rows_t,
            gs0, gs1, gs2, ism0, ism1, ism2, ism3, ism4, ism5, acc):
    cid = lax.axis_index("c")
    sid = lax.axis_index("s")
    wid = cid * NS + sid

    @pl.loop(0, ZR)
    def _(r):
        for c2 in range(F // 16):
            zbuf[r, pl.ds(c2 * 16, 16)] = jnp.zeros((16,), jnp.float32)

    @pl.loop(0, RPT // ZR)
    def _(i):
        pltpu.async_copy(zbuf, acc.at[pl.ds(sid * RPT + i * ZR, ZR)], gs0)

    @pl.loop(0, RPT // ZR)
    def _(i):
        pltpu.make_async_copy(zbuf, acc.at[pl.ds(sid * RPT, ZR)], gs0).wait()

    plsc.subcore_barrier()

    cbase = wid * NCHUNK
    slots = ((is0, ism0), (is1, ism1), (is2, ism2),
             (is3, ism3), (is4, ism4), (is5, ism5))
    rows = ((rb0, gs0), (rb1, gs1), (rb2, gs2))

    for q in range(3):
        pltpu.sync_copy(eidx_hbm.at[cbase + q], slots[q][0])
    for q in range(3, 6):
        pltpu.async_copy(eidx_hbm.at[cbase + q], slots[q][0], slots[q][1])
    for q in range(3):
        pltpu.async_copy(g_hbm.at[slots[q][0].at[0]], rows[q][0], rows[q][1])

    @pl.loop(0, (NCHUNK - 2) // 6)
    def _(t):
        c0 = 6 * t
        for q in range(6):
            idx_r, isem_r = slots[q]
            nidx_r, nisem_r = slots[(q + 3) % 6]
            rows_r, sem_r = rows[q % 3]
            pltpu.make_async_copy(g_hbm.at[idx_r.at[0]], rows_r, sem_r).wait()
            pltpu.sync_copy(rows_r, acc.at[idx_r.at[1]], add=True)
            pltpu.async_copy(eidx_hbm.at[cbase + c0 + q + 6], idx_r, isem_r)
            pltpu.make_async_copy(eidx_hbm.at[cbase], nidx_r, nisem_r).wait()
            pltpu.async_copy(g_hbm.at[nidx_r.at[0]], rows_r, sem_r)

    for q in range(2):
        idx_r, _ = slots[q]
        rows_r, sem_r = rows[q % 3]
        pltpu.make_async_copy(g_hbm.at[idx_r.at[0]], rows_r, sem_r).wait()
        pltpu.sync_copy(rows_r, acc.at[idx_r.at[1]], add=True)
    pltpu.make_async_copy(g_hbm.at[is2.at[0]], rb2, gs2).wait()
    for q in range(3, 6):
        pltpu.make_async_copy(eidx_hbm.at[cbase], slots[q][0], slots[q][1]).wait()

    pltpu.sync_copy(etail_hbm.at[wid], idx_t)
    pltpu.async_copy(g_hbm.at[idx_t.at[0]], rows_t, gs0).wait()
    pltpu.sync_copy(rows_t, acc.at[idx_t.at[1]], add=True)

    plsc.subcore_barrier()
    pltpu.sync_copy(
        acc.at[pl.ds(sid * RPT, RPT)], out_hbm.at[cid, pl.ds(sid * RPT, RPT)]
    )


BM = 2000

_CONTRACT_T = (((1,), (1,)), ((), ()))


def _tc1_body(d0, d1, x, w1, g_out, dinv_out):
    deg = d0[...] + d1[...] + 1.0
    dinv = lax.rsqrt(deg)
    h = lax.dot_general(x[...], w1[...], _CONTRACT_T,
                        preferred_element_type=jnp.float32)
    g_out[...] = dinv * h
    dinv_out[...] = dinv


_tc1 = pl.pallas_call(
    _tc1_body,
    grid=(N // BM,),
    in_specs=[
        pl.BlockSpec((BM, 1), lambda r: (r, 0)),
        pl.BlockSpec((BM, 1), lambda r: (r, 0)),
        pl.BlockSpec((BM, F), lambda r: (r, 0)),
        pl.BlockSpec((F, F), lambda r: (0, 0)),
    ],
    out_specs=[
        pl.BlockSpec((BM, F), lambda r: (r, 0)),
        pl.BlockSpec((BM, 1), lambda r: (r, 0)),
    ],
    out_shape=[
        jax.ShapeDtypeStruct((N, F), jnp.float32),
        jax.ShapeDtypeStruct((N, 1), jnp.float32),
    ],
)


def _tc2_body(acc, g1, dinv, b1, alpha, w2, g2_out):
    h = dinv[...] * (acc[0] + acc[1] + g1[...]) + b1[...]
    p = jnp.where(h > 0, h, alpha[...] * h)
    m = lax.dot_general(p, w2[...], _CONTRACT_T,
                        preferred_element_type=jnp.float32)
    g2_out[...] = dinv[...] * m


_tc2 = pl.pallas_call(
    _tc2_body,
    grid=(N // BM,),
    in_specs=[
        pl.BlockSpec((NC, BM, F), lambda r: (0, r, 0)),
        pl.BlockSpec((BM, F), lambda r: (r, 0)),
        pl.BlockSpec((BM, 1), lambda r: (r, 0)),
        pl.BlockSpec((1, F), lambda r: (0, 0)),
        pl.BlockSpec((1, F), lambda r: (0, 0)),
        pl.BlockSpec((F, F), lambda r: (0, 0)),
    ],
    out_specs=pl.BlockSpec((BM, F), lambda r: (r, 0)),
    out_shape=jax.ShapeDtypeStruct((N, F), jnp.float32),
)


def _tc3_body(acc, g2, dinv, b2, out):
    out[...] = dinv[...] * (acc[0] + acc[1] + g2[...]) + b2[...]


_tc3 = pl.pallas_call(
    _tc3_body,
    grid=(N // BM,),
    in_specs=[
        pl.BlockSpec((NC, BM, F), lambda r: (0, r, 0)),
        pl.BlockSpec((BM, F), lambda r: (r, 0)),
        pl.BlockSpec((BM, 1), lambda r: (r, 0)),
        pl.BlockSpec((1, F), lambda r: (0, 0)),
    ],
    out_specs=pl.BlockSpec((BM, F), lambda r: (r, 0)),
    out_shape=jax.ShapeDtypeStruct((N, F), jnp.float32),
)


def kernel(x, edge_index, W1, b1, W2, b2, alpha):
    ei = edge_index.reshape(2, NW, EPW)
    src_m = ei[0, :, : NCHUNK * K].reshape(NW, NCHUNK, K)
    dst_m = ei[1, :, : NCHUNK * K].reshape(NW, NCHUNK, K)
    eidx = jnp.concatenate(
        [jnp.stack([src_m, dst_m], axis=2).reshape(NW * NCHUNK, 2, K),
         jnp.zeros((4, 2, K), jnp.int32)])
    etail = ei[:, :, NCHUNK * K :].transpose(1, 0, 2)

    degp = _sc_deg(dst_m, etail)
    g1, dinv = _tc1(degp[0].reshape(NP, 1)[:N],
                    degp[1].reshape(NP, 1)[:N], x, W1)
    acc1 = _sc_agg(g1, eidx, etail)
    g2 = _tc2(acc1, g1, dinv, b1.reshape(1, F), alpha.reshape(1, F), W2)
    acc2 = _sc_agg(g2, eidx, etail)
    out = _tc3(acc2, g2, dinv, b2.reshape(1, F))
    return out
